# baseline (speedup 1.0000x reference)
import jax
import jax.numpy as jnp
from jax.experimental import pallas as pl
from jax.experimental.pallas import tpu as pltpu

_NCORES = 2


def _gru_kernel(x_ref, wi_ref, wh_ref, bi_ref, bhp_ref, out_ref):
    T, Bt, F = x_ref.shape
    H3 = wh_ref.shape[1]
    H = H3 // 3

    x2d = x_ref[...].reshape(T * Bt, F).astype(jnp.bfloat16)
    gi = (jnp.dot(x2d, wi_ref[...], preferred_element_type=jnp.float32)
          + bi_ref[...]).reshape(T, Bt, H3)

    w_h = wh_ref[...]
    b_hp = jnp.broadcast_to(bhp_ref[...], (Bt, H3))

    s0 = gi[0] + b_hp
    rz0 = jax.nn.sigmoid(s0[:, 0:2 * H])
    r0 = rz0[:, 0:H]
    z0 = rz0[:, H:2 * H]
    n0 = jnp.tanh(gi[0, :, 2 * H:] + r0 * b_hp[:, 2 * H:])
    h = (1.0 - z0) * n0
    out_ref[0] = h.astype(out_ref.dtype)

    for t in range(1, T):
        gh = jnp.dot(h.astype(jnp.bfloat16), w_h,
                     preferred_element_type=jnp.float32) + b_hp
        s = gi[t] + gh
        rz = jax.nn.sigmoid(s[:, 0:2 * H])
        r = rz[:, 0:H]
        z = rz[:, H:2 * H]
        n = jnp.tanh(gi[t, :, 2 * H:] + r * gh[:, 2 * H:])
        h = (1.0 - z) * n + z * h
        out_ref[t] = h.astype(out_ref.dtype)


def kernel(x, w_ih, w_hh, b_ih, b_hh):
    T, B, F = x.shape
    H = w_hh.shape[1]
    H3 = 3 * H

    B_pad = -(-B // (8 * _NCORES)) * (8 * _NCORES)
    if B_pad != B:
        x = jnp.pad(x, ((0, 0), (0, B_pad - B), (0, 0)))
    Bt = B_pad // _NCORES

    w_i = w_ih.T.astype(jnp.bfloat16)
    w_h = w_hh.T.astype(jnp.bfloat16)

    b_i = jnp.concatenate(
        [b_ih[0:2 * H] + b_hh[0:2 * H], b_ih[2 * H:H3]]).reshape(1, H3)
    b_hp = jnp.concatenate(
        [jnp.zeros((2 * H,), jnp.float32), b_hh[2 * H:H3]]).reshape(1, H3)

    flops = 2 * T * B_pad * F * H3 + 2 * (T - 1) * B_pad * H * H3
    bytes_accessed = 4 * T * B_pad * (F + H) + 2 * (F * H3 + H * H3) + 8 * H3

    out = pl.pallas_call(
        _gru_kernel,
        out_shape=jax.ShapeDtypeStruct((T, B_pad, H), jnp.float32),
        grid_spec=pltpu.PrefetchScalarGridSpec(
            num_scalar_prefetch=0,
            grid=(_NCORES,),
            in_specs=[
                pl.BlockSpec((T, Bt, F), lambda i: (0, i, 0)),
                pl.BlockSpec((F, H3), lambda i: (0, 0)),
                pl.BlockSpec((H, H3), lambda i: (0, 0)),
                pl.BlockSpec((1, H3), lambda i: (0, 0)),
                pl.BlockSpec((1, H3), lambda i: (0, 0)),
            ],
            out_specs=pl.BlockSpec((T, Bt, H), lambda i: (0, i, 0)),
        ),
        compiler_params=pltpu.CompilerParams(
            dimension_semantics=("parallel",),
        ),
        cost_estimate=pl.CostEstimate(
            flops=flops,
            transcendentals=3 * T * B_pad * H,
            bytes_accessed=bytes_accessed,
        ),
    )(x, w_i, w_h, b_i, b_hp)

    return out[:, :B, :]


# speedup vs baseline: 1.0939x; 1.0939x over previous
import jax
import jax.numpy as jnp
from jax.experimental import pallas as pl
from jax.experimental.pallas import tpu as pltpu

_NCORES = 1


def _gru_kernel(x_ref, wi_ref, wh_ref, bi_ref, bhp_ref, out_ref):
    T, Bt, F = x_ref.shape
    H3 = wh_ref.shape[1]
    H = H3 // 3

    x2d = x_ref[...].reshape(T * Bt, F).astype(jnp.bfloat16)
    gi = (jnp.dot(x2d, wi_ref[...], preferred_element_type=jnp.float32)
          + bi_ref[...]).reshape(T, Bt, H3)

    w_h = wh_ref[...]
    b_hp = jnp.broadcast_to(bhp_ref[...], (Bt, H3))

    s0 = gi[0] + b_hp
    rz0 = jax.nn.sigmoid(s0[:, 0:2 * H])
    r0 = rz0[:, 0:H]
    z0 = rz0[:, H:2 * H]
    n0 = jnp.tanh(gi[0, :, 2 * H:] + r0 * b_hp[:, 2 * H:])
    h = (1.0 - z0) * n0
    out_ref[0] = h.astype(out_ref.dtype)

    for t in range(1, T):
        gh = jnp.dot(h.astype(jnp.bfloat16), w_h,
                     preferred_element_type=jnp.float32) + b_hp
        s = gi[t] + gh
        rz = jax.nn.sigmoid(s[:, 0:2 * H])
        r = rz[:, 0:H]
        z = rz[:, H:2 * H]
        n = jnp.tanh(gi[t, :, 2 * H:] + r * gh[:, 2 * H:])
        h = (1.0 - z) * n + z * h
        out_ref[t] = h.astype(out_ref.dtype)


def kernel(x, w_ih, w_hh, b_ih, b_hh):
    T, B, F = x.shape
    H = w_hh.shape[1]
    H3 = 3 * H

    B_pad = -(-B // (8 * _NCORES)) * (8 * _NCORES)
    if B_pad != B:
        x = jnp.pad(x, ((0, 0), (0, B_pad - B), (0, 0)))
    Bt = B_pad // _NCORES

    w_i = w_ih.T.astype(jnp.bfloat16)
    w_h = w_hh.T.astype(jnp.bfloat16)

    b_i = jnp.concatenate(
        [b_ih[0:2 * H] + b_hh[0:2 * H], b_ih[2 * H:H3]]).reshape(1, H3)
    b_hp = jnp.concatenate(
        [jnp.zeros((2 * H,), jnp.float32), b_hh[2 * H:H3]]).reshape(1, H3)

    flops = 2 * T * B_pad * F * H3 + 2 * (T - 1) * B_pad * H * H3
    bytes_accessed = 4 * T * B_pad * (F + H) + 2 * (F * H3 + H * H3) + 8 * H3

    out = pl.pallas_call(
        _gru_kernel,
        out_shape=jax.ShapeDtypeStruct((T, B_pad, H), jnp.float32),
        grid_spec=pltpu.PrefetchScalarGridSpec(
            num_scalar_prefetch=0,
            grid=(_NCORES,),
            in_specs=[
                pl.BlockSpec((T, Bt, F), lambda i: (0, i, 0)),
                pl.BlockSpec((F, H3), lambda i: (0, 0)),
                pl.BlockSpec((H, H3), lambda i: (0, 0)),
                pl.BlockSpec((1, H3), lambda i: (0, 0)),
                pl.BlockSpec((1, H3), lambda i: (0, 0)),
            ],
            out_specs=pl.BlockSpec((T, Bt, H), lambda i: (0, i, 0)),
        ),
        compiler_params=pltpu.CompilerParams(
            dimension_semantics=("arbitrary",),
        ),
        cost_estimate=pl.CostEstimate(
            flops=flops,
            transcendentals=3 * T * B_pad * H,
            bytes_accessed=bytes_accessed,
        ),
    )(x, w_i, w_h, b_i, b_hp)

    return out[:, :B, :]


# speedup vs baseline: 1.3424x; 1.2271x over previous
import functools

import jax
import jax.numpy as jnp
from jax.experimental import pallas as pl
from jax.experimental.pallas import tpu as pltpu

_DN = (((1,), (1,)), ((), ()))


def _pick_tc(T):
    for c in (8, 6, 4, 3, 2):
        if T % c == 0:
            return c
    return 1


def _gru_kernel(x_hbm, wi_ref, wh_ref, bi_ref, bh_ref, out_hbm,
                xv, ov, in_sems, out_sems, *, Tc):
    T, B, F = xv.shape
    H3 = wi_ref.shape[0]
    H = H3 // 3
    NT = T // Tc

    for k in range(NT):
        pltpu.make_async_copy(x_hbm.at[pl.ds(k * Tc, Tc)],
                              xv.at[pl.ds(k * Tc, Tc)],
                              in_sems.at[k]).start()

    w_i = wi_ref[...].astype(jnp.bfloat16)
    w_h = wh_ref[...].astype(jnp.bfloat16)
    b_i = bi_ref[...]
    b_h = bh_ref[...]

    h = jnp.zeros((B, H), jnp.float32)
    gi = [None] * NT
    for k in range(NT + 1):
        if k < NT:
            pltpu.make_async_copy(xv.at[pl.ds(k * Tc, Tc)],
                                  xv.at[pl.ds(k * Tc, Tc)],
                                  in_sems.at[k]).wait()
            xc = xv[k * Tc:(k + 1) * Tc].reshape(Tc * B, F).astype(jnp.bfloat16)
            gi[k] = (jax.lax.dot_general(xc, w_i, _DN,
                                         preferred_element_type=jnp.float32)
                     + b_i).reshape(Tc, B, H3)
        if k >= 1:
            g = gi[k - 1]
            base = (k - 1) * Tc
            for i in range(Tc):
                gh = jax.lax.dot_general(h.astype(jnp.bfloat16), w_h, _DN,
                                         preferred_element_type=jnp.float32) + b_h
                s = g[i] + gh
                rz = jax.nn.sigmoid(s[:, :2 * H])
                r = rz[:, :H]
                z = rz[:, H:]
                n = jnp.tanh(g[i, :, 2 * H:] + r * gh[:, 2 * H:])
                h = (1.0 - z) * n + z * h
                ov[base + i] = h
            pltpu.make_async_copy(ov.at[pl.ds(base, Tc)],
                                  out_hbm.at[pl.ds(base, Tc)],
                                  out_sems.at[k - 1]).start()
            gi[k - 1] = None
    for k in range(NT):
        pltpu.make_async_copy(ov.at[pl.ds(k * Tc, Tc)],
                              out_hbm.at[pl.ds(k * Tc, Tc)],
                              out_sems.at[k]).wait()


def kernel(x, w_ih, w_hh, b_ih, b_hh):
    T, B, F = x.shape
    H = w_hh.shape[1]
    H3 = 3 * H

    B_pad = -(-B // 8) * 8
    if B_pad != B:
        x = jnp.pad(x, ((0, 0), (0, B_pad - B), (0, 0)))
    Tc = _pick_tc(T)
    NT = T // Tc

    b_i = b_ih.reshape(1, H3)
    b_h = b_hh.reshape(1, H3)

    flops = 2 * T * B_pad * F * H3 + 2 * T * B_pad * H * H3
    bytes_accessed = 4 * T * B_pad * (F + H) + 4 * (F * H3 + H * H3 + 2 * H3)

    out = pl.pallas_call(
        functools.partial(_gru_kernel, Tc=Tc),
        out_shape=jax.ShapeDtypeStruct((T, B_pad, H), jnp.float32),
        grid_spec=pltpu.PrefetchScalarGridSpec(
            num_scalar_prefetch=0,
            grid=(1,),
            in_specs=[
                pl.BlockSpec(memory_space=pl.ANY),
                pl.BlockSpec((H3, F), lambda i: (0, 0)),
                pl.BlockSpec((H3, H), lambda i: (0, 0)),
                pl.BlockSpec((1, H3), lambda i: (0, 0)),
                pl.BlockSpec((1, H3), lambda i: (0, 0)),
            ],
            out_specs=pl.BlockSpec(memory_space=pl.ANY),
            scratch_shapes=[
                pltpu.VMEM((T, B_pad, F), jnp.float32),
                pltpu.VMEM((T, B_pad, H), jnp.float32),
                pltpu.SemaphoreType.DMA((NT,)),
                pltpu.SemaphoreType.DMA((NT,)),
            ],
        ),
        compiler_params=pltpu.CompilerParams(
            dimension_semantics=("arbitrary",),
        ),
        cost_estimate=pl.CostEstimate(
            flops=flops,
            transcendentals=3 * T * B_pad * H,
            bytes_accessed=bytes_accessed,
        ),
    )(x, w_ih, w_hh, b_i, b_h)

    return out[:, :B, :]


# speedup vs baseline: 1.4575x; 1.0857x over previous
import functools

import jax
import jax.numpy as jnp
from jax.experimental import pallas as pl
from jax.experimental.pallas import tpu as pltpu


def _pick_tc(T):
    for c in (8, 6, 4, 3, 2):
        if T % c == 0:
            return c
    return 1


def _gru_kernel(x_hbm, wi_ref, wh_ref, bi_ref, bh_ref, out_hbm,
                xv, ov, wiv, whv, in_sems, out_sems, *, Tc):
    T, B, F = xv.shape
    H3 = wi_ref.shape[0]
    H = H3 // 3
    NT = T // Tc

    for k in range(NT):
        pltpu.make_async_copy(x_hbm.at[pl.ds(k * Tc, Tc)],
                              xv.at[pl.ds(k * Tc, Tc)],
                              in_sems.at[k]).start()

    lane = jax.lax.broadcasted_iota(jnp.int32, (1, H3), 1)
    scale = jnp.where(lane < 2 * H, 0.5, 1.0)
    wiv[...] = (wi_ref[...].T * scale).astype(jnp.bfloat16)
    whv[...] = (wh_ref[...].T * scale).astype(jnp.bfloat16)
    w_i = wiv[...]
    w_h = whv[...]
    b_i = jnp.where(lane < 2 * H,
                    0.5 * (bi_ref[...] + bh_ref[...]), bi_ref[...])
    bh_n = bh_ref[...][:, 2 * H:]

    h = jnp.zeros((B, H), jnp.float32)
    gi = [None] * NT
    pltpu.make_async_copy(xv.at[pl.ds(0, Tc)], xv.at[pl.ds(0, Tc)],
                          in_sems.at[0]).wait()
    for k in range(NT + 1):
        if k >= 1:
            g = gi[k - 1]
            base = (k - 1) * Tc
            for i in range(Tc):
                gh = jnp.dot(h.astype(jnp.bfloat16), w_h,
                             preferred_element_type=jnp.float32)
                trz = jnp.tanh(g[i, :, :2 * H] + gh[:, :2 * H])
                v = 0.5 * (gh[:, 2 * H:] + bh_n)
                n = jnp.tanh(g[i, :, 2 * H:] + v + trz[:, :H] * v)
                u = 0.5 * (h - n)
                h = n + u + trz[:, H:] * u
                ov[base + i] = h
            pltpu.make_async_copy(ov.at[pl.ds(base, Tc)],
                                  out_hbm.at[pl.ds(base, Tc)],
                                  out_sems.at[k - 1]).start()
            gi[k - 1] = None
        if k < NT:
            xc = xv[k * Tc:(k + 1) * Tc].reshape(Tc * B, F).astype(jnp.bfloat16)
            gi[k] = (jnp.dot(xc, w_i, preferred_element_type=jnp.float32)
                     + b_i).reshape(Tc, B, H3)
        if k + 1 < NT:
            pltpu.make_async_copy(xv.at[pl.ds((k + 1) * Tc, Tc)],
                                  xv.at[pl.ds((k + 1) * Tc, Tc)],
                                  in_sems.at[k + 1]).wait()
    for k in range(NT):
        pltpu.make_async_copy(ov.at[pl.ds(k * Tc, Tc)],
                              out_hbm.at[pl.ds(k * Tc, Tc)],
                              out_sems.at[k]).wait()


def kernel(x, w_ih, w_hh, b_ih, b_hh):
    T, B, F = x.shape
    H = w_hh.shape[1]
    H3 = 3 * H

    B_pad = -(-B // 8) * 8
    if B_pad != B:
        x = jnp.pad(x, ((0, 0), (0, B_pad - B), (0, 0)))
    Tc = _pick_tc(T)
    NT = T // Tc

    b_i = b_ih.reshape(1, H3)
    b_h = b_hh.reshape(1, H3)

    flops = 2 * T * B_pad * F * H3 + 2 * T * B_pad * H * H3
    bytes_accessed = 4 * T * B_pad * (F + H) + 4 * (F * H3 + H * H3 + 2 * H3)

    out = pl.pallas_call(
        functools.partial(_gru_kernel, Tc=Tc),
        out_shape=jax.ShapeDtypeStruct((T, B_pad, H), jnp.float32),
        grid_spec=pltpu.PrefetchScalarGridSpec(
            num_scalar_prefetch=0,
            grid=(1,),
            in_specs=[
                pl.BlockSpec(memory_space=pl.ANY),
                pl.BlockSpec((H3, F), lambda i: (0, 0)),
                pl.BlockSpec((H3, H), lambda i: (0, 0)),
                pl.BlockSpec((1, H3), lambda i: (0, 0)),
                pl.BlockSpec((1, H3), lambda i: (0, 0)),
            ],
            out_specs=pl.BlockSpec(memory_space=pl.ANY),
            scratch_shapes=[
                pltpu.VMEM((T, B_pad, F), jnp.float32),
                pltpu.VMEM((T, B_pad, H), jnp.float32),
                pltpu.VMEM((F, H3), jnp.bfloat16),
                pltpu.VMEM((H, H3), jnp.bfloat16),
                pltpu.SemaphoreType.DMA((NT,)),
                pltpu.SemaphoreType.DMA((NT,)),
            ],
        ),
        compiler_params=pltpu.CompilerParams(
            dimension_semantics=("arbitrary",),
        ),
        cost_estimate=pl.CostEstimate(
            flops=flops,
            transcendentals=3 * T * B_pad * H,
            bytes_accessed=bytes_accessed,
        ),
    )(x, w_ih, w_hh, b_i, b_h)

    return out if B_pad == B else out[:, :B, :]
